# Initial kernel scaffold; baseline (speedup 1.0000x reference)
import jax
import jax.numpy as jnp
from jax.experimental import pallas as pl


def kernel(x, conv0_w, conv0_b, conv1_w, conv1_b, fc0_w, fc0_b, out_w, out_b):
    raise NotImplementedError("write your pallas kernel here")


# speedup vs baseline: 30.2875x; 30.2875x over previous
import jax
import jax.numpy as jnp
from jax.experimental import pallas as pl
from jax.experimental.pallas import tpu as pltpu


def _fused_body(x_ref, w0_ref, b0_ref, w1_ref, b1_ref, f0_ref, fb_ref,
                ow_ref, ob_ref, o_ref):
    nb = x_ref.shape[3]
    xb = x_ref[...].astype(jnp.bfloat16)

    parts = [xb[:, kh:kh + 30, kw:kw + 30, :].reshape(3, 900 * nb)
             for kh in range(3) for kw in range(3)]
    slab0 = jnp.concatenate(parts, axis=0)
    y = jnp.dot(w0_ref[...], slab0, preferred_element_type=jnp.float32)
    y = jnp.maximum(y + b0_ref[...], 0.0)
    y = y.reshape(32, 30, 30, nb)

    y = y.reshape(32, 15, 2, 30, nb).max(axis=2)
    y = y.reshape(32, 15, 15, 2, nb).max(axis=3)
    p0 = y.astype(jnp.bfloat16)

    parts1 = [p0[:, kh:kh + 13, kw:kw + 13, :].reshape(32, 169 * nb)
              for kh in range(3) for kw in range(3)]
    slab1 = jnp.concatenate(parts1, axis=0)
    z = jnp.dot(w1_ref[...], slab1, preferred_element_type=jnp.float32)
    z = jnp.maximum(z + b1_ref[...], 0.0)
    z = z.reshape(64, 13, 13, nb)

    zh = jnp.concatenate(
        [z[:, 0:1], z[:, 1:13].reshape(64, 6, 2, 13, nb).max(axis=2)],
        axis=1)
    zw = jnp.concatenate(
        [zh[:, :, 0:1], zh[:, :, 1:13].reshape(64, 7, 6, 2, nb).max(axis=3)],
        axis=2)

    feat = zw.astype(jnp.bfloat16).reshape(3136, nb)
    h = jnp.dot(f0_ref[...], feat, preferred_element_type=jnp.float32)
    h = jnp.maximum(h + fb_ref[...], 0.0).astype(jnp.bfloat16)
    o = jnp.dot(ow_ref[...], h, preferred_element_type=jnp.float32)
    o_ref[...] = o + ob_ref[...]


def kernel(x, conv0_w, conv0_b, conv1_w, conv1_b, fc0_w, fc0_b, out_w, out_b):
    n = x.shape[0]
    nb = 128
    xt = jnp.transpose(x, (1, 2, 3, 0))
    w0r = jnp.transpose(conv0_w, (0, 2, 3, 1)).reshape(32, 27)
    w1r = jnp.transpose(conv1_w, (0, 2, 3, 1)).reshape(64, 288)
    args = (
        xt,
        w0r.astype(jnp.bfloat16), conv0_b.reshape(32, 1),
        w1r.astype(jnp.bfloat16), conv1_b.reshape(64, 1),
        fc0_w.T.astype(jnp.bfloat16), fc0_b.reshape(256, 1),
        out_w.T.astype(jnp.bfloat16), out_b.reshape(10, 1),
    )
    full = lambda s: pl.BlockSpec(s, lambda i: (0,) * len(s))
    out = pl.pallas_call(
        _fused_body,
        out_shape=jax.ShapeDtypeStruct((10, n), jnp.float32),
        grid=(n // nb,),
        in_specs=[
            pl.BlockSpec((3, 32, 32, nb), lambda i: (0, 0, 0, i)),
            full((32, 27)), full((32, 1)),
            full((64, 288)), full((64, 1)),
            full((256, 3136)), full((256, 1)),
            full((10, 256)), full((10, 1)),
        ],
        out_specs=pl.BlockSpec((10, nb), lambda i: (0, i)),
        compiler_params=pltpu.CompilerParams(
            dimension_semantics=("parallel",)),
    )(*args)
    return out.T


# speedup vs baseline: 63.0361x; 2.0813x over previous
import jax
import jax.numpy as jnp
from jax.experimental import pallas as pl
from jax.experimental.pallas import tpu as pltpu

_L = 128


def _fused_body(x_ref, w0_ref, w1_ref, f0_ref, fb_ref, ow_ref, ob_ref, o_ref):
    f32, bf16 = jnp.float32, jnp.bfloat16
    xb = x_ref[0].astype(bf16)

    s0 = 958
    parts0 = [xb[:, _L * (kh * 32 + kw): _L * (kh * 32 + kw + s0)]
              for kh in range(3) for kw in range(3)]
    parts0.append(jnp.ones((1, s0 * _L), bf16))
    slab0 = jnp.concatenate(parts0, axis=0)
    y = jnp.dot(w0_ref[...], slab0, preferred_element_type=f32)
    yb = jnp.maximum(y, 0.0).astype(bf16)

    a = jnp.maximum(yb[:, :-_L], yb[:, _L:])
    p = jnp.maximum(a[:, :-32 * _L], a[:, 32 * _L:])
    pc = jnp.concatenate(
        [p[:, _L * 64 * i: _L * (64 * i + 30)] for i in range(15)], axis=1)

    s1 = 385
    parts1 = [pc[:, _L * (kh * 30 + 2 * kw): _L * (kh * 30 + 2 * kw + s1)]
              for kh in range(3) for kw in range(3)]
    parts1.append(jnp.ones((1, s1 * _L), bf16))
    slab1 = jnp.concatenate(parts1, axis=0)
    z = jnp.dot(w1_ref[...], slab1, preferred_element_type=f32)
    zb = jnp.maximum(z, 0.0).astype(bf16)

    mw = jnp.maximum(zb[:, :-2 * _L], zb[:, 2 * _L:])
    mh = jnp.maximum(zb[:, :-30 * _L], zb[:, 30 * _L:])
    m2 = jnp.maximum(mw[:, :-30 * _L], mw[:, 30 * _L:])

    pieces = []
    for i in range(7):
        for j in range(7):
            if i == 0 and j == 0:
                src, t = zb, 0
            elif i == 0:
                src, t = mw, 2 * (2 * j - 1)
            elif j == 0:
                src, t = mh, (2 * i - 1) * 30
            else:
                src, t = m2, (2 * i - 1) * 30 + 2 * (2 * j - 1)
            pieces.append(src[:, _L * t: _L * (t + 1)])
    feat = jnp.concatenate(pieces, axis=0)

    h = jnp.dot(f0_ref[...], feat, preferred_element_type=f32)
    hb = jnp.maximum(h + fb_ref[...], 0.0).astype(bf16)
    o = jnp.dot(ow_ref[...], hb, preferred_element_type=f32)
    o_ref[...] = o + ob_ref[...]


def kernel(x, conv0_w, conv0_b, conv1_w, conv1_b, fc0_w, fc0_b, out_w, out_b):
    n = x.shape[0]
    nb = _L
    nblk = n // nb
    xr = (x.transpose(1, 2, 3, 0).reshape(3, 1024, nblk, nb)
          .transpose(2, 0, 1, 3).reshape(nblk, 3, 1024 * nb))
    w0r = jnp.transpose(conv0_w, (0, 2, 3, 1)).reshape(32, 27)
    w0p = jnp.concatenate([w0r, conv0_b[:, None]], axis=1).astype(jnp.bfloat16)
    w1r = jnp.transpose(conv1_w, (0, 2, 3, 1)).reshape(64, 288)
    w1p = jnp.concatenate([w1r, conv1_b[:, None]], axis=1).astype(jnp.bfloat16)
    f0p = (fc0_w.T.reshape(256, 64, 49).transpose(0, 2, 1)
           .reshape(256, 3136).astype(jnp.bfloat16))
    args = (
        xr, w0p, w1p,
        f0p, fc0_b.reshape(256, 1),
        out_w.T.astype(jnp.bfloat16), out_b.reshape(10, 1),
    )
    full = lambda s: pl.BlockSpec(s, lambda i: (0,) * len(s))
    out = pl.pallas_call(
        _fused_body,
        out_shape=jax.ShapeDtypeStruct((10, n), jnp.float32),
        grid=(nblk,),
        in_specs=[
            pl.BlockSpec((1, 3, 1024 * nb), lambda i: (i, 0, 0)),
            full((32, 28)), full((64, 289)),
            full((256, 3136)), full((256, 1)),
            full((10, 256)), full((10, 1)),
        ],
        out_specs=pl.BlockSpec((10, nb), lambda i: (0, i)),
        compiler_params=pltpu.CompilerParams(
            dimension_semantics=("parallel",)),
    )(*args)
    return out.T


# speedup vs baseline: 69.6148x; 1.1044x over previous
import jax
import jax.numpy as jnp
from jax.experimental import pallas as pl
from jax.experimental.pallas import tpu as pltpu

_L = 128


def _fused_body(x_ref, w0_ref, w1_ref, f0_ref, fb_ref, ow_ref, ob_ref, o_ref):
    f32, bf16 = jnp.float32, jnp.bfloat16
    xb = x_ref[0]

    s0 = 958
    parts0 = [xb[:, _L * (kh * 32 + kw): _L * (kh * 32 + kw + s0)]
              for kh in range(3) for kw in range(3)]
    parts0.append(jnp.ones((1, s0 * _L), bf16))
    slab0 = jnp.concatenate(parts0, axis=0)
    y = jnp.dot(w0_ref[...], slab0, preferred_element_type=f32)
    yb = jnp.maximum(y, 0.0).astype(bf16)

    a = jnp.maximum(yb[:, :-_L], yb[:, _L:])
    p = jnp.maximum(a[:, :-32 * _L], a[:, 32 * _L:])
    pc = jnp.concatenate(
        [p[:, _L * 64 * i: _L * (64 * i + 30)] for i in range(15)], axis=1)

    s1 = 385
    parts1 = [pc[:, _L * (kh * 30 + 2 * kw): _L * (kh * 30 + 2 * kw + s1)]
              for kh in range(3) for kw in range(3)]
    parts1.append(jnp.ones((1, s1 * _L), bf16))
    slab1 = jnp.concatenate(parts1, axis=0)
    z = jnp.dot(w1_ref[...], slab1, preferred_element_type=f32)
    zb = jnp.maximum(z, 0.0).astype(bf16)

    mw = jnp.maximum(zb[:, :-2 * _L], zb[:, 2 * _L:])
    mh = jnp.maximum(zb[:, :-30 * _L], zb[:, 30 * _L:])
    m2 = jnp.maximum(mw[:, :-30 * _L], mw[:, 30 * _L:])

    pieces = []
    for i in range(7):
        for j in range(7):
            if i == 0 and j == 0:
                src, t = zb, 0
            elif i == 0:
                src, t = mw, 2 * (2 * j - 1)
            elif j == 0:
                src, t = mh, (2 * i - 1) * 30
            else:
                src, t = m2, (2 * i - 1) * 30 + 2 * (2 * j - 1)
            pieces.append(src[:, _L * t: _L * (t + 1)])
    feat = jnp.concatenate(pieces, axis=0)

    h = jnp.dot(f0_ref[...], feat, preferred_element_type=f32)
    hb = jnp.maximum(h + fb_ref[...], 0.0).astype(bf16)
    o = jnp.dot(ow_ref[...], hb, preferred_element_type=f32)
    o_ref[...] = o + ob_ref[...]


def kernel(x, conv0_w, conv0_b, conv1_w, conv1_b, fc0_w, fc0_b, out_w, out_b):
    n = x.shape[0]
    nb = _L
    nblk = n // nb
    xr = (x.transpose(1, 2, 3, 0).reshape(3, 1024, nblk, nb)
          .transpose(2, 0, 1, 3).reshape(nblk, 3, 1024 * nb)
          .astype(jnp.bfloat16))
    w0r = jnp.transpose(conv0_w, (0, 2, 3, 1)).reshape(32, 27)
    w0p = jnp.concatenate([w0r, conv0_b[:, None]], axis=1).astype(jnp.bfloat16)
    w1r = jnp.transpose(conv1_w, (0, 2, 3, 1)).reshape(64, 288)
    w1p = jnp.concatenate([w1r, conv1_b[:, None]], axis=1).astype(jnp.bfloat16)
    f0p = (fc0_w.T.reshape(256, 64, 49).transpose(0, 2, 1)
           .reshape(256, 3136).astype(jnp.bfloat16))
    args = (
        xr, w0p, w1p,
        f0p, fc0_b.reshape(256, 1),
        out_w.T.astype(jnp.bfloat16), out_b.reshape(10, 1),
    )
    full = lambda s: pl.BlockSpec(s, lambda i: (0,) * len(s))
    out = pl.pallas_call(
        _fused_body,
        out_shape=jax.ShapeDtypeStruct((10, n), jnp.float32),
        grid=(nblk,),
        in_specs=[
            pl.BlockSpec((1, 3, 1024 * nb), lambda i: (i, 0, 0)),
            full((32, 28)), full((64, 289)),
            full((256, 3136)), full((256, 1)),
            full((10, 256)), full((10, 1)),
        ],
        out_specs=pl.BlockSpec((10, nb), lambda i: (0, i)),
        compiler_params=pltpu.CompilerParams(
            dimension_semantics=("parallel",)),
    )(*args)
    return out.T


# speedup vs baseline: 90.4317x; 1.2990x over previous
import jax
import jax.numpy as jnp
from jax.experimental import pallas as pl
from jax.experimental.pallas import tpu as pltpu

_L = 128


def _fused_body(x_ref, w0_ref, w1_ref, f0_ref, fb_ref, ow_ref, ob_ref, o_ref):
    f32, bf16 = jnp.float32, jnp.bfloat16
    xb = x_ref[...].reshape(3, 1024 * _L)

    s0 = 958
    parts0 = [xb[:, _L * (kh * 32 + kw): _L * (kh * 32 + kw + s0)]
              for kh in range(3) for kw in range(3)]
    parts0.append(jnp.ones((1, s0 * _L), bf16))
    slab0 = jnp.concatenate(parts0, axis=0)
    y = jnp.dot(w0_ref[...], slab0, preferred_element_type=f32)
    yb = jnp.maximum(y, 0.0).astype(bf16)

    a = jnp.maximum(yb[:, :-_L], yb[:, _L:])
    p = jnp.maximum(a[:, :-32 * _L], a[:, 32 * _L:])
    pc = jnp.concatenate(
        [p[:, _L * 64 * i: _L * (64 * i + 30)] for i in range(15)], axis=1)

    s1 = 385
    parts1 = [pc[:, _L * (kh * 30 + 2 * kw): _L * (kh * 30 + 2 * kw + s1)]
              for kh in range(3) for kw in range(3)]
    parts1.append(jnp.ones((1, s1 * _L), bf16))
    slab1 = jnp.concatenate(parts1, axis=0)
    z = jnp.dot(w1_ref[...], slab1, preferred_element_type=f32)
    zb = jnp.maximum(z, 0.0).astype(bf16)

    mw = jnp.maximum(zb[:, :-2 * _L], zb[:, 2 * _L:])
    mh = jnp.maximum(zb[:, :-30 * _L], zb[:, 30 * _L:])
    m2 = jnp.maximum(mw[:, :-30 * _L], mw[:, 30 * _L:])

    pieces = []
    for i in range(7):
        for j in range(7):
            if i == 0 and j == 0:
                src, t = zb, 0
            elif i == 0:
                src, t = mw, 2 * (2 * j - 1)
            elif j == 0:
                src, t = mh, (2 * i - 1) * 30
            else:
                src, t = m2, (2 * i - 1) * 30 + 2 * (2 * j - 1)
            pieces.append(src[:, _L * t: _L * (t + 1)])
    feat = jnp.concatenate(pieces, axis=0)

    h = jnp.dot(f0_ref[...], feat, preferred_element_type=f32)
    hb = jnp.maximum(h + fb_ref[...], 0.0).astype(bf16)
    o = jnp.dot(ow_ref[...], hb, preferred_element_type=f32)
    o_ref[...] = o + ob_ref[...]


def kernel(x, conv0_w, conv0_b, conv1_w, conv1_b, fc0_w, fc0_b, out_w, out_b):
    n = x.shape[0]
    nb = _L
    nblk = n // nb
    xr = (x.transpose(1, 2, 3, 0).reshape(3, 1024, n)
          .astype(jnp.bfloat16))
    w0r = jnp.transpose(conv0_w, (0, 2, 3, 1)).reshape(32, 27)
    w0p = jnp.concatenate([w0r, conv0_b[:, None]], axis=1).astype(jnp.bfloat16)
    w1r = jnp.transpose(conv1_w, (0, 2, 3, 1)).reshape(64, 288)
    w1p = jnp.concatenate([w1r, conv1_b[:, None]], axis=1).astype(jnp.bfloat16)
    f0p = (fc0_w.T.reshape(256, 64, 49).transpose(0, 2, 1)
           .reshape(256, 3136).astype(jnp.bfloat16))
    args = (
        xr, w0p, w1p,
        f0p, fc0_b.reshape(256, 1),
        out_w.T.astype(jnp.bfloat16), out_b.reshape(10, 1),
    )
    full = lambda s: pl.BlockSpec(s, lambda i: (0,) * len(s))
    out = pl.pallas_call(
        _fused_body,
        out_shape=jax.ShapeDtypeStruct((10, n), jnp.float32),
        grid=(nblk,),
        in_specs=[
            pl.BlockSpec((3, 1024, nb), lambda i: (0, 0, i)),
            full((32, 28)), full((64, 289)),
            full((256, 3136)), full((256, 1)),
            full((10, 256)), full((10, 1)),
        ],
        out_specs=pl.BlockSpec((10, nb), lambda i: (0, i)),
        compiler_params=pltpu.CompilerParams(
            dimension_semantics=("parallel",)),
    )(*args)
    return out.T


# speedup vs baseline: 112.1881x; 1.2406x over previous
import jax
import jax.numpy as jnp
from jax.experimental import pallas as pl
from jax.experimental.pallas import tpu as pltpu

_L = 128


def _fused_body(x_ref, w0_ref, w1_ref, f0_ref, fb_ref, ow_ref, ob_ref, o_ref):
    f32, bf16 = jnp.float32, jnp.bfloat16
    xb = x_ref[...].reshape(3, 1024 * _L)

    s0 = 958
    parts0 = [xb[:, _L * (kh * 32 + kw): _L * (kh * 32 + kw + s0)]
              for kh in range(3) for kw in range(3)]
    parts0.append(jnp.ones((1, s0 * _L), bf16))
    slab0 = jnp.concatenate(parts0, axis=0)
    y = jnp.dot(w0_ref[...], slab0, preferred_element_type=f32)
    yb = jnp.maximum(y, 0.0).astype(bf16)

    a = jnp.maximum(yb[:, :-_L], yb[:, _L:])
    p = jnp.maximum(a[:, :-32 * _L], a[:, 32 * _L:])
    pc = jnp.concatenate(
        [p[:, _L * (64 * i + 2 * j): _L * (64 * i + 2 * j + 1)]
         for i in range(15) for j in range(15)], axis=1)

    s1 = 193
    parts1 = [pc[:, _L * (kh * 15 + kw): _L * (kh * 15 + kw + s1)]
              for kh in range(3) for kw in range(3)]
    parts1.append(jnp.ones((1, s1 * _L), bf16))
    slab1 = jnp.concatenate(parts1, axis=0)
    z = jnp.dot(w1_ref[...], slab1, preferred_element_type=f32)
    zb = jnp.maximum(z, 0.0).astype(bf16)

    mw = jnp.maximum(zb[:, :-_L], zb[:, _L:])
    mh = jnp.maximum(zb[:, :-15 * _L], zb[:, 15 * _L:])
    m2 = jnp.maximum(mw[:, :-15 * _L], mw[:, 15 * _L:])

    pieces = []
    for i in range(7):
        for j in range(7):
            if i == 0 and j == 0:
                src, t = zb, 0
            elif i == 0:
                src, t = mw, 2 * j - 1
            elif j == 0:
                src, t = mh, (2 * i - 1) * 15
            else:
                src, t = m2, (2 * i - 1) * 15 + (2 * j - 1)
            pieces.append(src[:, _L * t: _L * (t + 1)])
    feat = jnp.concatenate(pieces, axis=0)

    h = jnp.dot(f0_ref[...], feat, preferred_element_type=f32)
    hb = jnp.maximum(h + fb_ref[...], 0.0).astype(bf16)
    o = jnp.dot(ow_ref[...], hb, preferred_element_type=f32)
    o_ref[...] = o + ob_ref[...]


def kernel(x, conv0_w, conv0_b, conv1_w, conv1_b, fc0_w, fc0_b, out_w, out_b):
    n = x.shape[0]
    nb = _L
    nblk = n // nb
    xr = (x.transpose(1, 2, 3, 0).reshape(3, 1024, n)
          .astype(jnp.bfloat16))
    w0r = jnp.transpose(conv0_w, (0, 2, 3, 1)).reshape(32, 27)
    w0p = jnp.concatenate([w0r, conv0_b[:, None]], axis=1).astype(jnp.bfloat16)
    w1r = jnp.transpose(conv1_w, (0, 2, 3, 1)).reshape(64, 288)
    w1p = jnp.concatenate([w1r, conv1_b[:, None]], axis=1).astype(jnp.bfloat16)
    f0p = (fc0_w.T.reshape(256, 64, 49).transpose(0, 2, 1)
           .reshape(256, 3136).astype(jnp.bfloat16))
    args = (
        xr, w0p, w1p,
        f0p, fc0_b.reshape(256, 1),
        out_w.T.astype(jnp.bfloat16), out_b.reshape(10, 1),
    )
    full = lambda s: pl.BlockSpec(s, lambda i: (0,) * len(s))
    out = pl.pallas_call(
        _fused_body,
        out_shape=jax.ShapeDtypeStruct((10, n), jnp.float32),
        grid=(nblk,),
        in_specs=[
            pl.BlockSpec((3, 1024, nb), lambda i: (0, 0, i)),
            full((32, 28)), full((64, 289)),
            full((256, 3136)), full((256, 1)),
            full((10, 256)), full((10, 1)),
        ],
        out_specs=pl.BlockSpec((10, nb), lambda i: (0, i)),
        compiler_params=pltpu.CompilerParams(
            dimension_semantics=("parallel",)),
    )(*args)
    return out.T


# speedup vs baseline: 116.8805x; 1.0418x over previous
import jax
import jax.numpy as jnp
from jax.experimental import pallas as pl
from jax.experimental.pallas import tpu as pltpu

_L = 256


def _fused_body(x_ref, w0_ref, w1_ref, f0_ref, fb_ref, ow_ref, ob_ref, o_ref):
    f32, bf16 = jnp.float32, jnp.bfloat16
    xb = x_ref[...].reshape(3, 1024 * _L)

    s0 = 958
    parts0 = [xb[:, _L * (kh * 32 + kw): _L * (kh * 32 + kw + s0)]
              for kh in range(3) for kw in range(3)]
    parts0.append(jnp.ones((1, s0 * _L), bf16))
    slab0 = jnp.concatenate(parts0, axis=0)
    y = jnp.dot(w0_ref[...], slab0, preferred_element_type=f32)
    yb = jnp.maximum(y, 0.0).astype(bf16)

    a = jnp.maximum(yb[:, :-_L], yb[:, _L:])
    p = jnp.maximum(a[:, :-32 * _L], a[:, 32 * _L:])
    pc = jnp.concatenate(
        [p[:, _L * (64 * i + 2 * j): _L * (64 * i + 2 * j + 1)]
         for i in range(15) for j in range(15)], axis=1)

    s1 = 193
    parts1 = [pc[:, _L * (kh * 15 + kw): _L * (kh * 15 + kw + s1)]
              for kh in range(3) for kw in range(3)]
    parts1.append(jnp.ones((1, s1 * _L), bf16))
    slab1 = jnp.concatenate(parts1, axis=0)
    z = jnp.dot(w1_ref[...], slab1, preferred_element_type=f32)
    zb = jnp.maximum(z, 0.0).astype(bf16)

    mw = jnp.maximum(zb[:, :-_L], zb[:, _L:])
    mh = jnp.maximum(zb[:, :-15 * _L], zb[:, 15 * _L:])
    m2 = jnp.maximum(mw[:, :-15 * _L], mw[:, 15 * _L:])

    pieces = []
    for i in range(7):
        for j in range(7):
            if i == 0 and j == 0:
                src, t = zb, 0
            elif i == 0:
                src, t = mw, 2 * j - 1
            elif j == 0:
                src, t = mh, (2 * i - 1) * 15
            else:
                src, t = m2, (2 * i - 1) * 15 + (2 * j - 1)
            pieces.append(src[:, _L * t: _L * (t + 1)])
    feat = jnp.concatenate(pieces, axis=0)

    h = jnp.dot(f0_ref[...], feat, preferred_element_type=f32)
    hb = jnp.maximum(h + fb_ref[...], 0.0).astype(bf16)
    o = jnp.dot(ow_ref[...], hb, preferred_element_type=f32)
    o_ref[...] = o + ob_ref[...]


def kernel(x, conv0_w, conv0_b, conv1_w, conv1_b, fc0_w, fc0_b, out_w, out_b):
    n = x.shape[0]
    nb = _L
    nblk = n // nb
    xr = (x.transpose(1, 2, 3, 0).reshape(3, 1024, n)
          .astype(jnp.bfloat16))
    w0r = jnp.transpose(conv0_w, (0, 2, 3, 1)).reshape(32, 27)
    w0p = jnp.concatenate([w0r, conv0_b[:, None]], axis=1).astype(jnp.bfloat16)
    w1r = jnp.transpose(conv1_w, (0, 2, 3, 1)).reshape(64, 288)
    w1p = jnp.concatenate([w1r, conv1_b[:, None]], axis=1).astype(jnp.bfloat16)
    f0p = (fc0_w.T.reshape(256, 64, 49).transpose(0, 2, 1)
           .reshape(256, 3136).astype(jnp.bfloat16))
    args = (
        xr, w0p, w1p,
        f0p, fc0_b.reshape(256, 1),
        out_w.T.astype(jnp.bfloat16), out_b.reshape(10, 1),
    )
    full = lambda s: pl.BlockSpec(s, lambda i: (0,) * len(s))
    out = pl.pallas_call(
        _fused_body,
        out_shape=jax.ShapeDtypeStruct((10, n), jnp.float32),
        grid=(nblk,),
        in_specs=[
            pl.BlockSpec((3, 1024, nb), lambda i: (0, 0, i)),
            full((32, 28)), full((64, 289)),
            full((256, 3136)), full((256, 1)),
            full((10, 256)), full((10, 1)),
        ],
        out_specs=pl.BlockSpec((10, nb), lambda i: (0, i)),
        compiler_params=pltpu.CompilerParams(
            dimension_semantics=("parallel",)),
    )(*args)
    return out.T
